# baseline (speedup 1.0000x reference)
import functools

import jax
import jax.numpy as jnp
from jax import lax
from jax.experimental import pallas as pl
from jax.experimental.pallas import tpu as pltpu
from jax.experimental.pallas import tpu_sc as plsc

K = 16
N = 2048
RB = 512
NC = 2
NS = 16
NW = NC * NS


def _knn_block(pts_r, ptsT_ref, xx_r, xx_c, idx_ref):
    rows = pts_r[0]
    cols = ptsT_ref[0]
    xxr = xx_r[0]
    xxc = xx_c[0]

    dot = (rows[:, 0:1] * cols[0:1, :]
           + rows[:, 1:2] * cols[1:2, :]
           + rows[:, 2:3] * cols[2:3, :])
    inner = -2.0 * dot
    pd = (-xxr.reshape(RB, 1) - inner) - xxc.reshape(1, N)

    iota = lax.broadcasted_iota(jnp.int32, (RB, N), 1)
    ninf = jnp.float32(-jnp.inf)
    boff = pl.program_id(0) * N
    cols_out = []
    am = None
    for t in range(K):
        if t:
            pd = jnp.where(iota == am[:, None], ninf, pd)
        am = jnp.argmax(pd, axis=1).astype(jnp.int32)
        cols_out.append(am + boff)
    idx_ref[0] = jnp.stack(cols_out, axis=1)


def _knn_pallas(pts):
    B = pts.shape[0]
    ptsT = jnp.swapaxes(pts, 1, 2)
    xx = jnp.sum(ptsT * ptsT, axis=1)[:, None, :]
    grid = (B, N // RB)
    return pl.pallas_call(
        _knn_block,
        grid=grid,
        in_specs=[
            pl.BlockSpec((1, RB, 3), lambda b, r: (b, r, 0)),
            pl.BlockSpec((1, 3, N), lambda b, r: (b, 0, 0)),
            pl.BlockSpec((1, 1, RB), lambda b, r: (b, 0, r)),
            pl.BlockSpec((1, 1, N), lambda b, r: (b, 0, 0)),
        ],
        out_specs=pl.BlockSpec((1, RB, K), lambda b, r: (b, r, 0)),
        out_shape=jax.ShapeDtypeStruct((B, N, K), jnp.int32),
    )(pts, ptsT, xx, xx)


TW = 128


SS = 1


def _gather_max_sc(table, idx_flat, KK, CW, M):
    n_chunks = (M * KK) // 128
    cpw = n_chunks // NW
    nss = cpw // SS
    OUTR = (SS * 128) // KK
    assert cpw * NW == n_chunks and nss * SS == cpw and nss % 2 == 0

    idx3d = idx_flat.reshape(n_chunks // SS, SS, 128)
    mesh = plsc.VectorSubcoreMesh(core_axis_name="c", subcore_axis_name="s",
                                  num_cores=NC, num_subcores=NS)

    @functools.partial(
        pl.kernel,
        out_type=jax.ShapeDtypeStruct((M, TW), jnp.float32),
        mesh=mesh,
        cost_estimate=pl.CostEstimate(
            flops=M * KK * TW,
            transcendentals=0,
            bytes_accessed=(M * KK * TW * 4) + M * TW * 4 + M * KK * 4,
        ),
        scratch_types=[
            pltpu.VMEM((SS, 128), jnp.int32),
            pltpu.VMEM((SS, 128), jnp.int32),
            pltpu.VMEM((SS * 128, TW), jnp.float32),
            pltpu.VMEM((SS * 128, TW), jnp.float32),
            pltpu.VMEM((OUTR, TW), jnp.float32),
            pltpu.SemaphoreType.DMA,
            pltpu.SemaphoreType.DMA,
        ],
    )
    def k(table_h, idx_h, out_h, idx_v0, idx_v1, rows_v0, rows_v1, out_v,
          sem0, sem1):
        wid = lax.axis_index("s") * NC + lax.axis_index("c")
        idx_b = (idx_v0, idx_v1)
        rows_b = (rows_v0, rows_v1)
        sem_b = (sem0, sem1)

        def issue(s, buf):
            gss = wid * nss + s
            pltpu.sync_copy(idx_h.at[gss], idx_b[buf])
            for j in range(SS):
                pltpu.async_copy(table_h.at[idx_b[buf].at[j]],
                                 rows_b[buf].at[pl.ds(j * 128, 128)],
                                 sem_b[buf])

        def drain(buf):
            for j in range(SS):
                pltpu.make_async_copy(table_h.at[idx_b[buf].at[j]],
                                      rows_b[buf].at[pl.ds(j * 128, 128)],
                                      sem_b[buf]).wait()

        def compute(s, buf):
            rows_v = rows_b[buf]
            row0 = (wid * cpw + s * SS) * (128 // KK)
            if KK == 1:
                pltpu.sync_copy(rows_v, out_h.at[pl.ds(row0, OUTR)])
            else:
                for i in range(OUTR):
                    for c in range(CW // 16):
                        sl = pl.ds(c * 16, 16)
                        vals = [rows_v[i * KK + j, sl] for j in range(KK)]
                        while len(vals) > 1:
                            vals = [jnp.maximum(vals[t], vals[t + 1])
                                    for t in range(0, len(vals) - 1, 2)] + (
                                        [vals[-1]] if len(vals) % 2 else [])
                        out_v[i, sl] = vals[0]
                pltpu.sync_copy(out_v, out_h.at[pl.ds(row0, OUTR)])

        issue(0, 0)

        def step(h, _):
            s0 = h * 2
            issue(s0 + 1, 1)
            drain(0)
            compute(s0, 0)

            @pl.when(h < nss // 2 - 1)
            def _():
                issue(s0 + 2, 0)

            drain(1)
            compute(s0 + 1, 1)
            return 0

        lax.fori_loop(0, nss // 2, step, 0)

    return k(table, idx3d)


RC = 1024


def _conv1_body(x0_ref, x1_ref, pts_ref, wa, ba, wb, bb, wc, bc, o_ref):
    x0 = x0_ref[:, :3]
    x1 = x1_ref[:, :3]
    outer = [x0[:, i:i + 1] * x1[:, j:j + 1] for i in range(3)
             for j in range(3)]
    xcov = jnp.concatenate([pts_ref[...]] + outer, axis=1)
    h = jnp.maximum(jnp.dot(xcov, wa[...]) + ba[...], 0.0)
    h = jnp.maximum(jnp.dot(h, wb[...]) + bb[...], 0.0)
    h = jnp.maximum(jnp.dot(h, wc[...]) + bc[...], 0.0)
    o_ref[...] = jnp.pad(h, ((0, 0), (0, TW - 64)))


def _conv1_pallas(nb01, pts_flat, W_m1a, b_m1a, W_m1b, b_m1b, W_m1c, b_m1c, M):
    wfull = lambda shape: pl.BlockSpec(shape, lambda r: (0, 0))
    return pl.pallas_call(
        _conv1_body,
        grid=(M // RC,),
        in_specs=[
            pl.BlockSpec((RC, TW), lambda r: (r, 0)),
            pl.BlockSpec((RC, TW), lambda r: (M // RC + r, 0)),
            pl.BlockSpec((RC, 3), lambda r: (r, 0)),
            wfull((12, 64)), wfull((1, 64)),
            wfull((64, 64)), wfull((1, 64)),
            wfull((64, 64)), wfull((1, 64)),
        ],
        out_specs=pl.BlockSpec((RC, TW), lambda r: (r, 0)),
        out_shape=jax.ShapeDtypeStruct((M, TW), jnp.float32),
    )(nb01, nb01, pts_flat, W_m1a.T.reshape(12, 64), b_m1a.reshape(1, 64),
      W_m1b.T.reshape(64, 64), b_m1b.reshape(1, 64),
      W_m1c.T.reshape(64, 64), b_m1c.reshape(1, 64))


def _conv2_body(x_ref, wl, bl, wc, bc, o_ref):
    x = x_ref[:, :64]
    h = jnp.dot(x, wl[...]) + bl[...]
    o_ref[...] = jnp.maximum(jnp.dot(h, wc[...]) + bc[...], 0.0)


def _conv2_pallas(xp, W_lin1, b_lin1, W_c1, b_c1, M):
    wfull = lambda shape: pl.BlockSpec(shape, lambda r: (0, 0))
    return pl.pallas_call(
        _conv2_body,
        grid=(M // RC,),
        in_specs=[
            pl.BlockSpec((RC, TW), lambda r: (r, 0)),
            wfull((64, 64)), wfull((1, 64)),
            wfull((64, 128)), wfull((1, 128)),
        ],
        out_specs=pl.BlockSpec((RC, 128), lambda r: (r, 0)),
        out_shape=jax.ShapeDtypeStruct((M, 128), jnp.float32),
    )(xp, W_lin1.T.reshape(64, 64), b_lin1.reshape(1, 64),
      W_c1.T.reshape(64, 128), b_c1.reshape(1, 128))


def _conv3_body(x_ref, wl, bl, wc, bc, o_ref):
    x = x_ref[...]
    h = jnp.dot(x, wl[...]) + bl[...]
    y = jnp.dot(h, wc[...]) + bc[...]
    part = jnp.max(y, axis=0, keepdims=True)[None]

    @pl.when(pl.program_id(1) == 0)
    def _():
        o_ref[...] = part

    @pl.when(pl.program_id(1) != 0)
    def _():
        o_ref[...] = jnp.maximum(o_ref[...], part)


def _conv3_pallas(x, W_lin2, b_lin2, W_c2, b_c2, B):
    wfull = lambda shape: pl.BlockSpec(shape, lambda b, r: (0, 0))
    return pl.pallas_call(
        _conv3_body,
        grid=(B, N // RC),
        in_specs=[
            pl.BlockSpec((RC, 128), lambda b, r: (b * (N // RC) + r, 0)),
            wfull((128, 128)), wfull((1, 128)),
            wfull((128, 1024)), wfull((1, 1024)),
        ],
        out_specs=pl.BlockSpec((1, 1, 1024), lambda b, r: (b, 0, 0)),
        out_shape=jax.ShapeDtypeStruct((B, 1, 1024), jnp.float32),
    )(x, W_lin2.T.reshape(128, 128), b_lin2.reshape(1, 128),
      W_c2.T.reshape(128, 1024), b_c2.reshape(1, 1024))[:, 0]


def _head_body(x_ref, wa, ba, wb, bb, o_ref):
    h = jnp.maximum(jnp.dot(x_ref[...], wa[...]) + ba[...], 0.0)
    o_ref[...] = jnp.dot(h, wb[...]) + bb[...]


def _head_pallas(x, W_m2a, b_m2a, W_m2b, b_m2b):
    B = x.shape[0]
    return pl.pallas_call(
        _head_body,
        out_shape=jax.ShapeDtypeStruct((B, 512), jnp.float32),
    )(x, W_m2a.T, b_m2a.reshape(1, 512), W_m2b.T, b_m2b.reshape(1, 512))


def _group_forward(pts, W_m1a, b_m1a, W_m1b, b_m1b, W_m1c, b_m1c,
                   W_lin1, b_lin1, W_c1, b_c1, W_lin2, b_lin2, W_c2, b_c2):
    B = pts.shape[0]
    M = B * N
    pts_flat = pts.reshape(M, 3)
    idx = _knn_pallas(pts)

    ptsw = jnp.pad(pts_flat, ((0, 0), (0, TW - 3)))
    idx01 = jnp.concatenate([idx[:, :, 0].reshape(M),
                             idx[:, :, 1].reshape(M)])
    nb01 = _gather_max_sc(ptsw, idx01, 1, TW, M * 2)

    x = _conv1_pallas(nb01, pts_flat, W_m1a, b_m1a, W_m1b, b_m1b,
                      W_m1c, b_m1c, M)

    idx_flat = idx.reshape(M * K)
    x = _gather_max_sc(x, idx_flat, K, 64, M)
    x = _conv2_pallas(x, W_lin1, b_lin1, W_c1, b_c1, M)
    x = _gather_max_sc(x, idx_flat, K, 128, M)
    return _conv3_pallas(x, W_lin2, b_lin2, W_c2, b_c2, B)


def kernel(pts, W_m1a, b_m1a, W_m1b, b_m1b, W_m1c, b_m1c, W_lin1, b_lin1,
           W_c1, b_c1, W_lin2, b_lin2, W_c2, b_c2, W_m2a, b_m2a, W_m2b, b_m2b):
    B = pts.shape[0]
    Wargs = (W_m1a, b_m1a, W_m1b, b_m1b, W_m1c, b_m1c, W_lin1, b_lin1,
             W_c1, b_c1, W_lin2, b_lin2, W_c2, b_c2)
    x = _group_forward(pts, *Wargs)
    return _head_pallas(x, W_m2a, b_m2a, W_m2b, b_m2b)[:, None, :]


# speedup vs baseline: 1.0087x; 1.0087x over previous
import functools

import jax
import jax.numpy as jnp
from jax import lax
from jax.experimental import pallas as pl
from jax.experimental.pallas import tpu as pltpu
from jax.experimental.pallas import tpu_sc as plsc

K = 16
N = 2048
RB = 512
NC = 2
NS = 16
NW = NC * NS


def _knn_block(pts_r, ptsT_ref, xx_r, xx_c, idx_ref):
    rows = pts_r[0]
    cols = ptsT_ref[0]
    xxr = xx_r[0]
    xxc = xx_c[0]

    dot = (rows[:, 0:1] * cols[0:1, :]
           + rows[:, 1:2] * cols[1:2, :]
           + rows[:, 2:3] * cols[2:3, :])
    inner = -2.0 * dot
    pd = (-xxr.reshape(RB, 1) - inner) - xxc.reshape(1, N)

    iota = lax.broadcasted_iota(jnp.int32, (RB, N), 1)
    ninf = jnp.float32(-jnp.inf)
    boff = pl.program_id(0) * N
    cols_out = []
    am = None
    for t in range(K):
        if t:
            pd = jnp.where(iota == am[:, None], ninf, pd)
        am = jnp.argmax(pd, axis=1).astype(jnp.int32)
        cols_out.append(am + boff)
    idx_ref[0] = jnp.stack(cols_out, axis=1)


def _knn_pallas(pts):
    B = pts.shape[0]
    ptsT = jnp.swapaxes(pts, 1, 2)
    xx = jnp.sum(ptsT * ptsT, axis=1)[:, None, :]
    grid = (B, N // RB)
    return pl.pallas_call(
        _knn_block,
        grid=grid,
        in_specs=[
            pl.BlockSpec((1, RB, 3), lambda b, r: (b, r, 0)),
            pl.BlockSpec((1, 3, N), lambda b, r: (b, 0, 0)),
            pl.BlockSpec((1, 1, RB), lambda b, r: (b, 0, r)),
            pl.BlockSpec((1, 1, N), lambda b, r: (b, 0, 0)),
        ],
        out_specs=pl.BlockSpec((1, RB, K), lambda b, r: (b, r, 0)),
        out_shape=jax.ShapeDtypeStruct((B, N, K), jnp.int32),
    )(pts, ptsT, xx, xx)


TW = 128


SS = 1


def _gather_max_sc(table, idx_flat, KK, CW, M):
    n_chunks = (M * KK) // 128
    cpw = n_chunks // NW
    nss = cpw // SS
    OUTR = (SS * 128) // KK
    assert cpw * NW == n_chunks and nss * SS == cpw and nss % 2 == 0

    idx3d = idx_flat.reshape(n_chunks // SS, SS, 128)
    mesh = plsc.VectorSubcoreMesh(core_axis_name="c", subcore_axis_name="s",
                                  num_cores=NC, num_subcores=NS)

    @functools.partial(
        pl.kernel,
        out_type=jax.ShapeDtypeStruct((M, TW), jnp.float32),
        mesh=mesh,
        cost_estimate=pl.CostEstimate(
            flops=M * KK * TW,
            transcendentals=0,
            bytes_accessed=(M * KK * TW * 4) + M * TW * 4 + M * KK * 4,
        ),
        scratch_types=[
            pltpu.VMEM((SS, 128), jnp.int32),
            pltpu.VMEM((SS, 128), jnp.int32),
            pltpu.VMEM((SS * 128, TW), jnp.float32),
            pltpu.VMEM((SS * 128, TW), jnp.float32),
            pltpu.VMEM((OUTR, TW), jnp.float32),
            pltpu.SemaphoreType.DMA,
            pltpu.SemaphoreType.DMA,
        ],
    )
    def k(table_h, idx_h, out_h, idx_v0, idx_v1, rows_v0, rows_v1, out_v,
          sem0, sem1):
        wid = lax.axis_index("s") * NC + lax.axis_index("c")
        idx_b = (idx_v0, idx_v1)
        rows_b = (rows_v0, rows_v1)
        sem_b = (sem0, sem1)

        def issue(s, buf):
            gss = wid * nss + s
            pltpu.sync_copy(idx_h.at[gss], idx_b[buf])
            for j in range(SS):
                pltpu.async_copy(table_h.at[idx_b[buf].at[j]],
                                 rows_b[buf].at[pl.ds(j * 128, 128)],
                                 sem_b[buf])

        def drain(buf):
            for j in range(SS):
                pltpu.make_async_copy(table_h.at[idx_b[buf].at[j]],
                                      rows_b[buf].at[pl.ds(j * 128, 128)],
                                      sem_b[buf]).wait()

        def compute(s, buf):
            rows_v = rows_b[buf]
            row0 = (wid * cpw + s * SS) * (128 // KK)
            if KK == 1:
                pltpu.sync_copy(rows_v, out_h.at[pl.ds(row0, OUTR)])
            else:
                for i in range(OUTR):
                    for c in range(CW // 16):
                        sl = pl.ds(c * 16, 16)
                        vals = [rows_v[i * KK + j, sl] for j in range(KK)]
                        while len(vals) > 1:
                            vals = [jnp.maximum(vals[t], vals[t + 1])
                                    for t in range(0, len(vals) - 1, 2)] + (
                                        [vals[-1]] if len(vals) % 2 else [])
                        out_v[i, sl] = vals[0]
                pltpu.sync_copy(out_v, out_h.at[pl.ds(row0, OUTR)])

        issue(0, 0)

        def step(h, _):
            s0 = h * 2
            issue(s0 + 1, 1)
            drain(0)
            compute(s0, 0)

            @pl.when(h < nss // 2 - 1)
            def _():
                issue(s0 + 2, 0)

            drain(1)
            compute(s0 + 1, 1)
            return 0

        lax.fori_loop(0, nss // 2, step, 0)

    return k(table, idx3d)


RC = 1024


def _conv1_body(x0_ref, x1_ref, pts_ref, wa, ba, wb, bb, wc, bc, o_ref):
    x0 = x0_ref[:, :3]
    x1 = x1_ref[:, :3]
    outer = [x0[:, i:i + 1] * x1[:, j:j + 1] for i in range(3)
             for j in range(3)]
    xcov = jnp.concatenate([pts_ref[...]] + outer, axis=1)
    h = jnp.maximum(jnp.dot(xcov, wa[...]) + ba[...], 0.0)
    h = jnp.maximum(jnp.dot(h, wb[...]) + bb[...], 0.0)
    h = jnp.maximum(jnp.dot(h, wc[...]) + bc[...], 0.0)
    o_ref[...] = jnp.pad(h, ((0, 0), (0, TW - 64)))


def _conv1_pallas(nb01, pts_flat, W_m1a, b_m1a, W_m1b, b_m1b, W_m1c, b_m1c, M):
    wfull = lambda shape: pl.BlockSpec(shape, lambda r: (0, 0))
    return pl.pallas_call(
        _conv1_body,
        grid=(M // RC,),
        in_specs=[
            pl.BlockSpec((RC, TW), lambda r: (r, 0)),
            pl.BlockSpec((RC, TW), lambda r: (M // RC + r, 0)),
            pl.BlockSpec((RC, 3), lambda r: (r, 0)),
            wfull((12, 64)), wfull((1, 64)),
            wfull((64, 64)), wfull((1, 64)),
            wfull((64, 64)), wfull((1, 64)),
        ],
        out_specs=pl.BlockSpec((RC, TW), lambda r: (r, 0)),
        out_shape=jax.ShapeDtypeStruct((M, TW), jnp.float32),
    )(nb01, nb01, pts_flat, W_m1a.T.reshape(12, 64), b_m1a.reshape(1, 64),
      W_m1b.T.reshape(64, 64), b_m1b.reshape(1, 64),
      W_m1c.T.reshape(64, 64), b_m1c.reshape(1, 64))


def _conv2_body(x_ref, wl, bl, wc, bc, o_ref):
    x = x_ref[:, :64]
    h = jnp.dot(x, wl[...]) + bl[...]
    o_ref[...] = jnp.maximum(jnp.dot(h, wc[...]) + bc[...], 0.0)


def _conv2_pallas(xp, W_lin1, b_lin1, W_c1, b_c1, M):
    wfull = lambda shape: pl.BlockSpec(shape, lambda r: (0, 0))
    return pl.pallas_call(
        _conv2_body,
        grid=(M // RC,),
        in_specs=[
            pl.BlockSpec((RC, TW), lambda r: (r, 0)),
            wfull((64, 64)), wfull((1, 64)),
            wfull((64, 128)), wfull((1, 128)),
        ],
        out_specs=pl.BlockSpec((RC, 128), lambda r: (r, 0)),
        out_shape=jax.ShapeDtypeStruct((M, 128), jnp.float32),
    )(xp, W_lin1.T.reshape(64, 64), b_lin1.reshape(1, 64),
      W_c1.T.reshape(64, 128), b_c1.reshape(1, 128))


def _conv3_body(x_ref, wl, bl, wc, bc, o_ref):
    x = x_ref[...]
    h = jnp.dot(x, wl[...]) + bl[...]
    y = jnp.dot(h, wc[...]) + bc[...]
    part = jnp.max(y, axis=0, keepdims=True)[None]

    @pl.when(pl.program_id(1) == 0)
    def _():
        o_ref[...] = part

    @pl.when(pl.program_id(1) != 0)
    def _():
        o_ref[...] = jnp.maximum(o_ref[...], part)


def _conv3_pallas(x, W_lin2, b_lin2, W_c2, b_c2, B):
    wfull = lambda shape: pl.BlockSpec(shape, lambda b, r: (0, 0))
    return pl.pallas_call(
        _conv3_body,
        grid=(B, N // RC),
        in_specs=[
            pl.BlockSpec((RC, 128), lambda b, r: (b * (N // RC) + r, 0)),
            wfull((128, 128)), wfull((1, 128)),
            wfull((128, 1024)), wfull((1, 1024)),
        ],
        out_specs=pl.BlockSpec((1, 1, 1024), lambda b, r: (b, 0, 0)),
        out_shape=jax.ShapeDtypeStruct((B, 1, 1024), jnp.float32),
    )(x, W_lin2.T.reshape(128, 128), b_lin2.reshape(1, 128),
      W_c2.T.reshape(128, 1024), b_c2.reshape(1, 1024))[:, 0]


def _head_body(x_ref, wa, ba, wb, bb, o_ref):
    h = jnp.maximum(jnp.dot(x_ref[...], wa[...]) + ba[...], 0.0)
    o_ref[...] = jnp.dot(h, wb[...]) + bb[...]


def _head_pallas(x, W_m2a, b_m2a, W_m2b, b_m2b):
    B = x.shape[0]
    return pl.pallas_call(
        _head_body,
        out_shape=jax.ShapeDtypeStruct((B, 512), jnp.float32),
    )(x, W_m2a.T, b_m2a.reshape(1, 512), W_m2b.T, b_m2b.reshape(1, 512))


def _group_forward(pts, W_m1a, b_m1a, W_m1b, b_m1b, W_m1c, b_m1c,
                   W_lin1, b_lin1, W_c1, b_c1, W_lin2, b_lin2, W_c2, b_c2):
    B = pts.shape[0]
    M = B * N
    pts_flat = pts.reshape(M, 3)
    idx = _knn_pallas(pts)

    ptsw = jnp.pad(pts_flat, ((0, 0), (0, TW - 3)))
    idx01 = jnp.concatenate([idx[:, :, 0].reshape(M),
                             idx[:, :, 1].reshape(M)])
    nb01 = _gather_max_sc(ptsw, idx01, 1, TW, M * 2)

    x = _conv1_pallas(nb01, pts_flat, W_m1a, b_m1a, W_m1b, b_m1b,
                      W_m1c, b_m1c, M)

    idx_flat = idx.reshape(M * K)
    x = _gather_max_sc(x, idx_flat, K, 64, M)
    x = _conv2_pallas(x, W_lin1, b_lin1, W_c1, b_c1, M)
    x = _gather_max_sc(x, idx_flat, K, 128, M)
    return _conv3_pallas(x, W_lin2, b_lin2, W_c2, b_c2, B)


def kernel(pts, W_m1a, b_m1a, W_m1b, b_m1b, W_m1c, b_m1c, W_lin1, b_lin1,
           W_c1, b_c1, W_lin2, b_lin2, W_c2, b_c2, W_m2a, b_m2a, W_m2b, b_m2b):
    B = pts.shape[0]
    Wargs = (W_m1a, b_m1a, W_m1b, b_m1b, W_m1c, b_m1c, W_lin1, b_lin1,
             W_c1, b_c1, W_lin2, b_lin2, W_c2, b_c2)
    h0 = _group_forward(pts[:B // 2], *Wargs)
    h1 = _group_forward(pts[B // 2:], *Wargs)
    x = jnp.concatenate([h0, h1], axis=0)
    return _head_pallas(x, W_m2a, b_m2a, W_m2b, b_m2b)[:, None, :]


# speedup vs baseline: 1.0224x; 1.0136x over previous
import functools

import jax
import jax.numpy as jnp
from jax import lax
from jax.experimental import pallas as pl
from jax.experimental.pallas import tpu as pltpu
from jax.experimental.pallas import tpu_sc as plsc

K = 16
N = 2048
RB = 512
NC = 2
NS = 16
NW = NC * NS


def _knn_block(pts_r, ptsT_ref, xx_r, xx_c, idx_ref):
    rows = pts_r[0]
    cols = ptsT_ref[0]
    xxr = xx_r[0]
    xxc = xx_c[0]

    dot = jnp.dot(rows, cols, preferred_element_type=jnp.float32)
    inner = -2.0 * dot
    pd = (-xxr.reshape(RB, 1) - inner) - xxc.reshape(1, N)

    iota = lax.broadcasted_iota(jnp.int32, (RB, N), 1)
    ninf = jnp.float32(-jnp.inf)
    boff = pl.program_id(0) * N
    cols_out = []
    am = None
    for t in range(K):
        if t:
            pd = jnp.where(iota == am[:, None], ninf, pd)
        am = jnp.argmax(pd, axis=1).astype(jnp.int32)
        cols_out.append(am + boff)
    idx_ref[0] = jnp.stack(cols_out, axis=1)


def _knn_pallas(pts):
    B = pts.shape[0]
    ptsT = jnp.swapaxes(pts, 1, 2)
    xx = jnp.sum(ptsT * ptsT, axis=1)[:, None, :]
    grid = (B, N // RB)
    return pl.pallas_call(
        _knn_block,
        grid=grid,
        in_specs=[
            pl.BlockSpec((1, RB, 3), lambda b, r: (b, r, 0)),
            pl.BlockSpec((1, 3, N), lambda b, r: (b, 0, 0)),
            pl.BlockSpec((1, 1, RB), lambda b, r: (b, 0, r)),
            pl.BlockSpec((1, 1, N), lambda b, r: (b, 0, 0)),
        ],
        out_specs=pl.BlockSpec((1, RB, K), lambda b, r: (b, r, 0)),
        out_shape=jax.ShapeDtypeStruct((B, N, K), jnp.int32),
    )(pts, ptsT, xx, xx)


TW = 128


SS = 1


def _gather_max_sc(table, idx_flat, KK, CW, M):
    n_chunks = (M * KK) // 128
    cpw = n_chunks // NW
    nss = cpw // SS
    OUTR = (SS * 128) // KK
    assert cpw * NW == n_chunks and nss * SS == cpw and nss % 2 == 0

    idx3d = idx_flat.reshape(n_chunks // SS, SS, 128)
    mesh = plsc.VectorSubcoreMesh(core_axis_name="c", subcore_axis_name="s",
                                  num_cores=NC, num_subcores=NS)

    @functools.partial(
        pl.kernel,
        out_type=jax.ShapeDtypeStruct((M, TW), jnp.float32),
        mesh=mesh,
        cost_estimate=pl.CostEstimate(
            flops=M * KK * TW,
            transcendentals=0,
            bytes_accessed=(M * KK * TW * 4) + M * TW * 4 + M * KK * 4,
        ),
        scratch_types=[
            pltpu.VMEM((SS, 128), jnp.int32),
            pltpu.VMEM((SS, 128), jnp.int32),
            pltpu.VMEM((SS * 128, TW), jnp.float32),
            pltpu.VMEM((SS * 128, TW), jnp.float32),
            pltpu.VMEM((OUTR, TW), jnp.float32),
            pltpu.SemaphoreType.DMA,
            pltpu.SemaphoreType.DMA,
        ],
    )
    def k(table_h, idx_h, out_h, idx_v0, idx_v1, rows_v0, rows_v1, out_v,
          sem0, sem1):
        wid = lax.axis_index("s") * NC + lax.axis_index("c")
        idx_b = (idx_v0, idx_v1)
        rows_b = (rows_v0, rows_v1)
        sem_b = (sem0, sem1)

        def issue(s, buf):
            gss = wid * nss + s
            pltpu.sync_copy(idx_h.at[gss], idx_b[buf])
            for j in range(SS):
                pltpu.async_copy(table_h.at[idx_b[buf].at[j]],
                                 rows_b[buf].at[pl.ds(j * 128, 128)],
                                 sem_b[buf])

        def drain(buf):
            for j in range(SS):
                pltpu.make_async_copy(table_h.at[idx_b[buf].at[j]],
                                      rows_b[buf].at[pl.ds(j * 128, 128)],
                                      sem_b[buf]).wait()

        def compute(s, buf):
            rows_v = rows_b[buf]
            row0 = (wid * cpw + s * SS) * (128 // KK)
            if KK == 1:
                pltpu.sync_copy(rows_v, out_h.at[pl.ds(row0, OUTR)])
            else:
                for i in range(OUTR):
                    for c in range(CW // 16):
                        sl = pl.ds(c * 16, 16)
                        vals = [rows_v[i * KK + j, sl] for j in range(KK)]
                        while len(vals) > 1:
                            vals = [jnp.maximum(vals[t], vals[t + 1])
                                    for t in range(0, len(vals) - 1, 2)] + (
                                        [vals[-1]] if len(vals) % 2 else [])
                        out_v[i, sl] = vals[0]
                pltpu.sync_copy(out_v, out_h.at[pl.ds(row0, OUTR)])

        issue(0, 0)

        def step(h, _):
            s0 = h * 2
            issue(s0 + 1, 1)
            drain(0)
            compute(s0, 0)

            @pl.when(h < nss // 2 - 1)
            def _():
                issue(s0 + 2, 0)

            drain(1)
            compute(s0 + 1, 1)
            return 0

        lax.fori_loop(0, nss // 2, step, 0)

    return k(table, idx3d)


RC = 1024


def _conv1_body(x0_ref, x1_ref, pts_ref, wa, ba, wb, bb, wc, bc, o_ref):
    x0 = x0_ref[:, :3]
    x1 = x1_ref[:, :3]
    outer = [x0[:, i:i + 1] * x1[:, j:j + 1] for i in range(3)
             for j in range(3)]
    xcov = jnp.concatenate([pts_ref[...]] + outer, axis=1)
    h = jnp.maximum(jnp.dot(xcov, wa[...]) + ba[...], 0.0)
    h = jnp.maximum(jnp.dot(h, wb[...]) + bb[...], 0.0)
    h = jnp.maximum(jnp.dot(h, wc[...]) + bc[...], 0.0)
    o_ref[...] = jnp.pad(h, ((0, 0), (0, TW - 64)))


def _conv1_pallas(nb01, pts_flat, W_m1a, b_m1a, W_m1b, b_m1b, W_m1c, b_m1c, M):
    wfull = lambda shape: pl.BlockSpec(shape, lambda r: (0, 0))
    return pl.pallas_call(
        _conv1_body,
        grid=(M // RC,),
        in_specs=[
            pl.BlockSpec((RC, TW), lambda r: (r, 0)),
            pl.BlockSpec((RC, TW), lambda r: (M // RC + r, 0)),
            pl.BlockSpec((RC, 3), lambda r: (r, 0)),
            wfull((12, 64)), wfull((1, 64)),
            wfull((64, 64)), wfull((1, 64)),
            wfull((64, 64)), wfull((1, 64)),
        ],
        out_specs=pl.BlockSpec((RC, TW), lambda r: (r, 0)),
        out_shape=jax.ShapeDtypeStruct((M, TW), jnp.float32),
    )(nb01, nb01, pts_flat, W_m1a.T.reshape(12, 64), b_m1a.reshape(1, 64),
      W_m1b.T.reshape(64, 64), b_m1b.reshape(1, 64),
      W_m1c.T.reshape(64, 64), b_m1c.reshape(1, 64))


def _conv2_body(x_ref, wl, bl, wc, bc, o_ref):
    x = x_ref[:, :64]
    h = jnp.dot(x, wl[...]) + bl[...]
    o_ref[...] = jnp.maximum(jnp.dot(h, wc[...]) + bc[...], 0.0)


def _conv2_pallas(xp, W_lin1, b_lin1, W_c1, b_c1, M):
    wfull = lambda shape: pl.BlockSpec(shape, lambda r: (0, 0))
    return pl.pallas_call(
        _conv2_body,
        grid=(M // RC,),
        in_specs=[
            pl.BlockSpec((RC, TW), lambda r: (r, 0)),
            wfull((64, 64)), wfull((1, 64)),
            wfull((64, 128)), wfull((1, 128)),
        ],
        out_specs=pl.BlockSpec((RC, 128), lambda r: (r, 0)),
        out_shape=jax.ShapeDtypeStruct((M, 128), jnp.float32),
    )(xp, W_lin1.T.reshape(64, 64), b_lin1.reshape(1, 64),
      W_c1.T.reshape(64, 128), b_c1.reshape(1, 128))


def _conv3_body(x_ref, wl, bl, wc, bc, o_ref):
    x = x_ref[...]
    h = jnp.dot(x, wl[...]) + bl[...]
    y = jnp.dot(h, wc[...]) + bc[...]
    part = jnp.max(y, axis=0, keepdims=True)[None]

    @pl.when(pl.program_id(1) == 0)
    def _():
        o_ref[...] = part

    @pl.when(pl.program_id(1) != 0)
    def _():
        o_ref[...] = jnp.maximum(o_ref[...], part)


def _conv3_pallas(x, W_lin2, b_lin2, W_c2, b_c2, B):
    wfull = lambda shape: pl.BlockSpec(shape, lambda b, r: (0, 0))
    return pl.pallas_call(
        _conv3_body,
        grid=(B, N // RC),
        in_specs=[
            pl.BlockSpec((RC, 128), lambda b, r: (b * (N // RC) + r, 0)),
            wfull((128, 128)), wfull((1, 128)),
            wfull((128, 1024)), wfull((1, 1024)),
        ],
        out_specs=pl.BlockSpec((1, 1, 1024), lambda b, r: (b, 0, 0)),
        out_shape=jax.ShapeDtypeStruct((B, 1, 1024), jnp.float32),
    )(x, W_lin2.T.reshape(128, 128), b_lin2.reshape(1, 128),
      W_c2.T.reshape(128, 1024), b_c2.reshape(1, 1024))[:, 0]


def _head_body(x_ref, wa, ba, wb, bb, o_ref):
    h = jnp.maximum(jnp.dot(x_ref[...], wa[...]) + ba[...], 0.0)
    o_ref[...] = jnp.dot(h, wb[...]) + bb[...]


def _head_pallas(x, W_m2a, b_m2a, W_m2b, b_m2b):
    B = x.shape[0]
    return pl.pallas_call(
        _head_body,
        out_shape=jax.ShapeDtypeStruct((B, 512), jnp.float32),
    )(x, W_m2a.T, b_m2a.reshape(1, 512), W_m2b.T, b_m2b.reshape(1, 512))


def _group_forward(pts, W_m1a, b_m1a, W_m1b, b_m1b, W_m1c, b_m1c,
                   W_lin1, b_lin1, W_c1, b_c1, W_lin2, b_lin2, W_c2, b_c2):
    B = pts.shape[0]
    M = B * N
    pts_flat = pts.reshape(M, 3)
    idx = _knn_pallas(pts)

    ptsw = jnp.pad(pts_flat, ((0, 0), (0, TW - 3)))
    idx01 = jnp.concatenate([idx[:, :, 0].reshape(M),
                             idx[:, :, 1].reshape(M)])
    nb01 = _gather_max_sc(ptsw, idx01, 1, TW, M * 2)

    x = _conv1_pallas(nb01, pts_flat, W_m1a, b_m1a, W_m1b, b_m1b,
                      W_m1c, b_m1c, M)

    idx_flat = idx.reshape(M * K)
    x = _gather_max_sc(x, idx_flat, K, 64, M)
    x = _conv2_pallas(x, W_lin1, b_lin1, W_c1, b_c1, M)
    x = _gather_max_sc(x, idx_flat, K, 128, M)
    return _conv3_pallas(x, W_lin2, b_lin2, W_c2, b_c2, B)


def kernel(pts, W_m1a, b_m1a, W_m1b, b_m1b, W_m1c, b_m1c, W_lin1, b_lin1,
           W_c1, b_c1, W_lin2, b_lin2, W_c2, b_c2, W_m2a, b_m2a, W_m2b, b_m2b):
    B = pts.shape[0]
    Wargs = (W_m1a, b_m1a, W_m1b, b_m1b, W_m1c, b_m1c, W_lin1, b_lin1,
             W_c1, b_c1, W_lin2, b_lin2, W_c2, b_c2)
    h0 = _group_forward(pts[:B // 2], *Wargs)
    h1 = _group_forward(pts[B // 2:], *Wargs)
    x = jnp.concatenate([h0, h1], axis=0)
    return _head_pallas(x, W_m2a, b_m2a, W_m2b, b_m2b)[:, None, :]
